# baseline (speedup 1.0000x reference)
import functools

import jax
import jax.numpy as jnp
from jax import lax
from jax.experimental import pallas as pl
from jax.experimental.pallas import tpu as pltpu
from jax.experimental.pallas import tpu_sc as plsc

_B, _S = 1024, 200
_L = 4
_VL, _VT = 100000, 512
_D = 32
_N = _B * _S
_NW = 32
_C = 128
_R = _L * _C
_NIB = _R // 128
_TW = _N // _NW
_NCHW = _TW // _C


def _make_kernel():
    mesh = plsc.VectorSubcoreMesh(core_axis_name="c", subcore_axis_name="s")

    @functools.partial(
        pl.kernel,
        mesh=mesh,
        out_type=jax.ShapeDtypeStruct((_N, 2 * _D), jnp.float32),
        compiler_params=pltpu.CompilerParams(use_tc_tiling_on_sc=False),
        scratch_types=[
            pltpu.VMEM((2, _R), jnp.int32),
            pltpu.VMEM((2, _R), jnp.int32),
            pltpu.VMEM((2, _R, _D), jnp.float32),
            pltpu.VMEM((2, _R, _D), jnp.float32),
            pltpu.VMEM((2, _C, 2 * _D), jnp.float32),
            pltpu.VMEM((2 * _L, 16), jnp.float32),
            pltpu.SemaphoreType.DMA,
            pltpu.SemaphoreType.DMA,
            pltpu.SemaphoreType.DMA,
            pltpu.SemaphoreType.DMA,
            pltpu.SemaphoreType.DMA,
            pltpu.SemaphoreType.DMA,
        ],
    )
    def k(x_hbm, t_hbm, loc_hbm, time_hbm, w_hbm, out_hbm,
          idx_x, idx_t, rows_x, rows_t, out_v, w_v,
          sem_i0, sem_i1, sem_g0, sem_g1, sem_o0, sem_o1):
        wid = lax.axis_index("s") * 2 + lax.axis_index("c")
        g0 = wid * _NCHW
        sem_i = [sem_i0, sem_i1]
        sem_g = [sem_g0, sem_g1]
        sem_o = [sem_o0, sem_o1]

        pltpu.sync_copy(w_hbm, w_v)
        ws = [w_v[j] for j in range(2 * _L)]

        lane_lvl = lax.rem(lax.iota(jnp.int32, 16), jnp.int32(_L))
        off_x = lane_lvl * jnp.int32(_VL)
        off_t = lane_lvl * jnp.int32(_VT)

        def stage_idx(par, g):
            pltpu.async_copy(x_hbm.at[pl.ds(g * _R, _R)],
                             idx_x.at[par], sem_i[par])
            pltpu.async_copy(t_hbm.at[pl.ds(g * _R, _R)],
                             idx_t.at[par], sem_i[par])

        def wait_idx(par):
            pltpu.make_async_copy(
                x_hbm.at[pl.ds(0, _R)], idx_x.at[par], sem_i[par]).wait()
            pltpu.make_async_copy(
                t_hbm.at[pl.ds(0, _R)], idx_t.at[par], sem_i[par]).wait()

        def add_offsets(par):
            @plsc.parallel_loop(0, _R, step=16, unroll=4)
            def _(i):
                idx_x[par, pl.ds(i, 16)] = idx_x[par, pl.ds(i, 16)] + off_x
                idx_t[par, pl.ds(i, 16)] = idx_t[par, pl.ds(i, 16)] + off_t

        def issue_gathers(par):
            for q in range(_NIB):
                pltpu.async_copy(
                    loc_hbm.at[idx_x.at[par, pl.ds(q * 128, 128)]],
                    rows_x.at[par, pl.ds(q * 128, 128)], sem_g[par])
                pltpu.async_copy(
                    time_hbm.at[idx_t.at[par, pl.ds(q * 128, 128)]],
                    rows_t.at[par, pl.ds(q * 128, 128)], sem_g[par])

        def wait_gathers(par):
            for q in range(_NIB):
                pltpu.make_async_copy(
                    loc_hbm.at[pl.ds(0, 128)],
                    rows_x.at[par, pl.ds(q * 128, 128)], sem_g[par]).wait()
                pltpu.make_async_copy(
                    loc_hbm.at[pl.ds(0, 128)],
                    rows_t.at[par, pl.ds(q * 128, 128)], sem_g[par]).wait()

        def compute(par):
            @plsc.parallel_loop(0, _C, unroll=4)
            def _(c):
                r = c * _L
                for p in range(2):
                    sl = p * 16
                    a = ws[0] * rows_x[par, r, pl.ds(sl, 16)]
                    for j in range(1, _L):
                        a = a + ws[j] * rows_x[par, r + j, pl.ds(sl, 16)]
                    out_v[par, c, pl.ds(sl, 16)] = a
                    b = ws[_L] * rows_t[par, r, pl.ds(sl, 16)]
                    for j in range(1, _L):
                        b = b + ws[_L + j] * rows_t[par, r + j, pl.ds(sl, 16)]
                    out_v[par, c, pl.ds(_D + sl, 16)] = b

        def out_slice(g):
            return out_hbm.at[pl.ds(g * _C, _C), :]

        stage_idx(0, g0)
        stage_idx(1, g0 + 1)
        wait_idx(0)
        add_offsets(0)
        issue_gathers(0)

        def super_body(i, carry):
            for par in range(2):
                g = g0 + 2 * i + par
                nxt = 1 - par
                wait_gathers(par)

                @pl.when(i < _NCHW // 2 - 1)
                def _prefetch_idx():
                    stage_idx(par, g + 2)

                def _launch_next():
                    wait_idx(nxt)
                    add_offsets(nxt)
                    issue_gathers(nxt)

                if par == 0:
                    _launch_next()
                else:
                    pl.when(i < _NCHW // 2 - 1)(_launch_next)

                @pl.when(i > 0)
                def _drain_out():
                    pltpu.make_async_copy(
                        out_v.at[par], out_slice(g - 2), sem_o[par]).wait()

                compute(par)
                pltpu.async_copy(out_v.at[par], out_slice(g), sem_o[par])
            return carry

        lax.fori_loop(0, _NCHW // 2, super_body, 0)

        last = g0 + _NCHW - 2
        pltpu.make_async_copy(out_v.at[0], out_slice(last), sem_o[0]).wait()
        pltpu.make_async_copy(out_v.at[1], out_slice(last + 1),
                              sem_o[1]).wait()

    return k


_k = _make_kernel()


def kernel(x, t, loc_tables, time_tables, x_weights, t_weights):
    xf = x.reshape(_N * _L).astype(jnp.int32)
    tf = t.reshape(_N * _L).astype(jnp.int32)
    loc_flat = loc_tables.reshape(_L * _VL, _D)
    time_flat = time_tables.reshape(_L * _VT, _D)
    w_all = jnp.broadcast_to(
        jnp.concatenate([x_weights, t_weights])[:, None], (2 * _L, 16))
    out = _k(xf, tf, loc_flat, time_flat, w_all)
    return out.reshape(_B, _S, 2 * _D)
